# Initial kernel scaffold; baseline (speedup 1.0000x reference)
import jax
import jax.numpy as jnp
from jax.experimental import pallas as pl


def kernel(x, edge_index, W1, b1, W2, b2, W3, b3):
    raise NotImplementedError("write your pallas kernel here")


# speedup vs baseline: 2.2073x; 2.2073x over previous
import functools

import jax
import jax.numpy as jnp
from jax import lax
from jax.experimental import pallas as pl
from jax.experimental.pallas import tpu as pltpu
from jax.experimental.pallas import tpu_sc as plsc

N = 10000
NP = 10240
D = 128
E = 320000
NC = 2
NS = 16
NW = NC * NS
EPW = E // NW
KP = 128
CHP = 80
EP = KP * CHP
NQ = 4
QR = NP // NQ
QT = QR // NS
ACC = QR + 8
EPT = E // NS
CH2 = 160
EP2 = CH2 * KP
HRT = NP // NS


def _sc_hist_body(dst_hbm, out_hbm, dst_v, ones_v, zbuf_v, hist_sh):
    c = lax.axis_index("c")
    s = lax.axis_index("s")
    wid = c * NS + s

    def fill_ones(i, t):
        ones_v[pl.ds(i * 16, 16)] = jnp.ones((16,), jnp.float32)
        return t

    lax.fori_loop(0, KP // 16, fill_ones, 0)

    def fill_zero(i, t):
        zbuf_v[pl.ds(i * 16, 16)] = jnp.zeros((16,), jnp.float32)
        return t

    lax.fori_loop(0, HRT // 16, fill_zero, 0)

    hbase = pl.multiple_of(s * HRT, 8)
    pltpu.sync_copy(dst_hbm.at[wid], dst_v)
    pltpu.sync_copy(zbuf_v, hist_sh.at[pl.ds(hbase, HRT)])
    plsc.subcore_barrier()

    def chunk(i, t):
        pltpu.sync_copy(ones_v, hist_sh.at[dst_v.at[i]], add=True)
        return t

    lax.fori_loop(0, CHP, chunk, 0)
    plsc.subcore_barrier()

    pltpu.sync_copy(hist_sh.at[pl.ds(hbase, HRT)], zbuf_v)
    pltpu.sync_copy(zbuf_v, out_hbm.at[c, s, 0])


def _sc_agg_body(h_hbm, src_hbm, dstq_hbm, out_hbm, src_v, dst_v,
                 rows_v, buf_v, rbuf_v, acc_sh, sem):
    c = lax.axis_index("c")
    s = lax.axis_index("s")

    def zrow(i, t):
        def zcol(j, u):
            buf_v[i, pl.ds(j * 16, 16)] = jnp.zeros((16,), jnp.float32)
            return u

        return lax.fori_loop(0, D // 16, zcol, t)

    lax.fori_loop(0, QT, zrow, 0)

    abase = pl.multiple_of(s * QT, 8)
    pltpu.sync_copy(src_hbm.at[s], src_v)

    for p in range(NQ // NC):
        qbase = pl.multiple_of(jnp.int32(2 * p * QR) + c * QR, QR)
        obase = pl.multiple_of(qbase + s * QT, 8)
        pltpu.sync_copy(dstq_hbm.at[p, c, s], dst_v)
        pltpu.sync_copy(buf_v, acc_sh.at[pl.ds(abase, QT)])
        plsc.subcore_barrier()

        def chunk(i, t):
            pltpu.async_copy(h_hbm.at[src_v.at[i]], rows_v, sem).wait()
            pltpu.sync_copy(rows_v, acc_sh.at[dst_v.at[i]], add=True)
            return t

        lax.fori_loop(0, CH2, chunk, 0)
        plsc.subcore_barrier()

        pltpu.sync_copy(acc_sh.at[pl.ds(abase, QT)], rbuf_v)
        pltpu.sync_copy(rbuf_v, out_hbm.at[pl.ds(obase, QT)])
        plsc.subcore_barrier()


@functools.cache
def _build_sc_kernels():
    mesh = plsc.VectorSubcoreMesh(core_axis_name="c", subcore_axis_name="s")
    sc_hist = functools.partial(
        pl.kernel,
        mesh=mesh,
        out_type=jax.ShapeDtypeStruct((NC, NS, 1, HRT), jnp.float32),
        scratch_types=[
            pltpu.VMEM((CHP, KP), jnp.int32),
            pltpu.VMEM((KP,), jnp.float32),
            pltpu.VMEM((HRT,), jnp.float32),
            pltpu.VMEM_SHARED((NP,), jnp.float32),
        ],
    )(_sc_hist_body)
    sc_agg = functools.partial(
        pl.kernel,
        mesh=mesh,
        out_type=jax.ShapeDtypeStruct((NP, D), jnp.float32),
        scratch_types=[
            pltpu.VMEM((CH2, KP), jnp.int32),
            pltpu.VMEM((CH2, KP), jnp.int32),
            pltpu.VMEM((KP, D), jnp.float32),
            pltpu.VMEM((QT, D), jnp.float32),
            pltpu.VMEM((QT, D), jnp.float32),
            pltpu.VMEM_SHARED((ACC, D), jnp.float32),
            pltpu.SemaphoreType.DMA,
        ],
    )(_sc_agg_body)
    return sc_hist, sc_agg


def _tc_pre_body(x_ref, w_ref, hist_ref, hp_ref, dis_ref):
    hist = jnp.squeeze(hist_ref[...], axis=2)
    deg = hist[0] + hist[1] + 1.0
    dis3 = lax.rsqrt(deg)[:, :, None]
    dis = jnp.broadcast_to(dis3, (NS, HRT, D)).reshape(NP, D)
    h = jnp.dot(x_ref[...], w_ref[...], preferred_element_type=jnp.float32)
    hp_ref[...] = h * dis
    dis_ref[...] = dis


_pre_call = pl.pallas_call(
    _tc_pre_body,
    out_shape=(jax.ShapeDtypeStruct((NP, D), jnp.float32),
               jax.ShapeDtypeStruct((NP, D), jnp.float32)),
)


def _tc_mid_body(p_ref, hp_ref, dis_ref, b_ref, w_ref, out_ref):
    z = jnp.maximum(
        (p_ref[...] + hp_ref[...]) * dis_ref[...] + b_ref[...], 0.0)
    out_ref[...] = (jnp.dot(z, w_ref[...], preferred_element_type=jnp.float32)
                    * dis_ref[...])


_mid_call = pl.pallas_call(
    _tc_mid_body,
    out_shape=jax.ShapeDtypeStruct((NP, D), jnp.float32),
)


def _tc_post_body(p_ref, hp_ref, dis_ref, b_ref, out_ref):
    out_ref[...] = jnp.maximum(
        (p_ref[...] + hp_ref[...]) * dis_ref[...] + b_ref[...], 0.0)


_post_call = pl.pallas_call(
    _tc_post_body,
    out_shape=jax.ShapeDtypeStruct((NP, D), jnp.float32),
)


def _pad_worker_lists(ei):
    src_w = ei[0].reshape(NW, EPW)
    dst_w = ei[1].reshape(NW, EPW)
    pad_d = jnp.full((NW, EP - EPW), NP - 1, jnp.int32)
    dst3h = jnp.concatenate([dst_w, pad_d], axis=1).reshape(NW, CHP, KP)

    src_t = ei[0].reshape(NS, EPT)
    dst_t = ei[1].reshape(NS, EPT)
    pad_s2 = jnp.zeros((NS, EP2 - EPT), jnp.int32)
    pad_d2 = jnp.full((NS, EP2 - EPT), NP, jnp.int32)
    src2 = jnp.concatenate([src_t, pad_s2], axis=1).reshape(NS, CH2, KP)
    dst2 = jnp.concatenate([dst_t, pad_d2], axis=1)
    dstq = jnp.stack(
        [jnp.stack(
            [jnp.minimum((dst2 - (2 * p + c) * QR).astype(jnp.uint32),
                         jnp.uint32(QR)).astype(jnp.int32).reshape(
                             NS, CH2, KP)
             for c in range(NC)])
         for p in range(NQ // NC)])
    return dst3h, src2, dstq


def kernel(x, edge_index, W1, b1, W2, b2, W3, b3):
    ei = edge_index.astype(jnp.int32)
    dst3h, src2, dstq = _pad_worker_lists(ei)
    xp = jnp.pad(x, ((0, NP - N), (0, 0)))
    _sc_hist, _sc_agg = _build_sc_kernels()
    hist = _sc_hist(dst3h)
    hp1, dis = _pre_call(xp, W1, hist)
    p1 = _sc_agg(hp1, src2, dstq)
    hp2 = _mid_call(p1, hp1, dis, b1.reshape(1, D), W2)
    p2 = _sc_agg(hp2, src2, dstq)
    hp3 = _mid_call(p2, hp2, dis, b2.reshape(1, D), W3)
    p3 = _sc_agg(hp3, src2, dstq)
    return _post_call(p3, hp3, dis, b3.reshape(1, D))[:N]


# speedup vs baseline: 2.2586x; 1.0232x over previous
import functools

import jax
import jax.numpy as jnp
from jax import lax
from jax.experimental import pallas as pl
from jax.experimental.pallas import tpu as pltpu
from jax.experimental.pallas import tpu_sc as plsc

N = 10000
NP = 10240
D = 128
E = 320000
NC = 2
NS = 16
NW = NC * NS
EPW = E // NW
KP = 128
CHP = 80
EP = KP * CHP
NQ = 4
QR = NP // NQ
QT = QR // NS
ACC = QR + 8
EPT = E // NS
CH2 = 160
EP2 = CH2 * KP
HRT = NP // NS


def _sc_hist_body(dst_hbm, out_hbm, dst_v, ones_v, zbuf_v, hist_sh):
    c = lax.axis_index("c")
    s = lax.axis_index("s")
    wid = c * NS + s

    def fill_ones(i, t):
        ones_v[pl.ds(i * 16, 16)] = jnp.ones((16,), jnp.float32)
        return t

    lax.fori_loop(0, KP // 16, fill_ones, 0)

    def fill_zero(i, t):
        zbuf_v[pl.ds(i * 16, 16)] = jnp.zeros((16,), jnp.float32)
        return t

    lax.fori_loop(0, HRT // 16, fill_zero, 0)

    hbase = pl.multiple_of(s * HRT, 8)
    pltpu.sync_copy(dst_hbm.at[wid], dst_v)
    pltpu.sync_copy(zbuf_v, hist_sh.at[pl.ds(hbase, HRT)])
    plsc.subcore_barrier()

    def chunk(i, t):
        pltpu.sync_copy(ones_v, hist_sh.at[dst_v.at[i]], add=True)
        return t

    lax.fori_loop(0, CHP, chunk, 0)
    plsc.subcore_barrier()

    pltpu.sync_copy(hist_sh.at[pl.ds(hbase, HRT)], zbuf_v)
    pltpu.sync_copy(zbuf_v, out_hbm.at[c, s, 0])


def _sc_agg_body(h_hbm, src_hbm, dstq_hbm, out_hbm, src_v, dst_v,
                 rows_v, buf_v, rbuf_v, acc_sh, sem):
    c = lax.axis_index("c")
    s = lax.axis_index("s")

    def zrow(i, t):
        def zcol(j, u):
            buf_v[i, pl.ds(j * 16, 16)] = jnp.zeros((16,), jnp.float32)
            return u

        return lax.fori_loop(0, D // 16, zcol, t)

    lax.fori_loop(0, QT, zrow, 0)

    abase = pl.multiple_of(s * QT, 8)
    pltpu.sync_copy(src_hbm.at[s], src_v)

    for p in range(NQ // NC):
        qbase = pl.multiple_of(jnp.int32(2 * p * QR) + c * QR, QR)
        obase = pl.multiple_of(qbase + s * QT, 8)
        pltpu.sync_copy(dstq_hbm.at[p, c, s], dst_v)
        pltpu.sync_copy(buf_v, acc_sh.at[pl.ds(abase, QT)])
        plsc.subcore_barrier()

        def chunk(i, t):
            pltpu.async_copy(h_hbm.at[src_v.at[i]], rows_v, sem).wait()
            pltpu.sync_copy(rows_v, acc_sh.at[dst_v.at[i]], add=True)
            return t

        lax.fori_loop(0, CH2, chunk, 0)
        plsc.subcore_barrier()

        pltpu.sync_copy(acc_sh.at[pl.ds(abase, QT)], rbuf_v)
        pltpu.sync_copy(rbuf_v, out_hbm.at[pl.ds(obase, QT)])
        plsc.subcore_barrier()


@functools.cache
def _build_sc_kernels():
    mesh = plsc.VectorSubcoreMesh(core_axis_name="c", subcore_axis_name="s")
    sc_hist = functools.partial(
        pl.kernel,
        mesh=mesh,
        out_type=jax.ShapeDtypeStruct((NC, NS, 1, HRT), jnp.float32),
        scratch_types=[
            pltpu.VMEM((CHP, KP), jnp.int32),
            pltpu.VMEM((KP,), jnp.float32),
            pltpu.VMEM((HRT,), jnp.float32),
            pltpu.VMEM_SHARED((NP,), jnp.float32),
        ],
    )(_sc_hist_body)
    sc_agg = functools.partial(
        pl.kernel,
        mesh=mesh,
        out_type=jax.ShapeDtypeStruct((NP, D), jnp.float32),
        scratch_types=[
            pltpu.VMEM((CH2, KP), jnp.int32),
            pltpu.VMEM((CH2, KP), jnp.int32),
            pltpu.VMEM((KP, D), jnp.float32),
            pltpu.VMEM((QT, D), jnp.float32),
            pltpu.VMEM((QT, D), jnp.float32),
            pltpu.VMEM_SHARED((ACC, D), jnp.float32),
            pltpu.SemaphoreType.DMA,
        ],
    )(_sc_agg_body)
    return sc_hist, sc_agg


def _tc_pre_body(x_ref, w_ref, hist_ref, hp_ref, dis_ref):
    hist = jnp.squeeze(hist_ref[...], axis=2)
    deg = hist[0] + hist[1] + 1.0
    dis3 = lax.rsqrt(deg)[:, :, None]
    dis = jnp.broadcast_to(dis3, (NS, HRT, D)).reshape(NP, D)
    h = jnp.dot(x_ref[...], w_ref[...], preferred_element_type=jnp.float32)
    hp_ref[...] = h * dis
    dis_ref[...] = dis


_pre_call = pl.pallas_call(
    _tc_pre_body,
    out_shape=(jax.ShapeDtypeStruct((NP, D), jnp.float32),
               jax.ShapeDtypeStruct((NP, D), jnp.float32)),
)


def _tc_mid_body(p_ref, hp_ref, dis_ref, b_ref, w_ref, out_ref):
    z = jnp.maximum(
        (p_ref[...] + hp_ref[...]) * dis_ref[...] + b_ref[...], 0.0)
    out_ref[...] = (jnp.dot(z, w_ref[...], preferred_element_type=jnp.float32)
                    * dis_ref[...])


_mid_call = pl.pallas_call(
    _tc_mid_body,
    out_shape=jax.ShapeDtypeStruct((NP, D), jnp.float32),
)


def _tc_post_body(p_ref, hp_ref, dis_ref, b_ref, out_ref):
    out_ref[...] = jnp.maximum(
        (p_ref[...] + hp_ref[...]) * dis_ref[...] + b_ref[...], 0.0)


_post_call = pl.pallas_call(
    _tc_post_body,
    out_shape=jax.ShapeDtypeStruct((NP, D), jnp.float32),
)


def _pad_worker_lists(ei):
    src_w = ei[0].reshape(NW, EPW)
    dst_w = ei[1].reshape(NW, EPW)
    pad_d = jnp.full((NW, EP - EPW), NP - 1, jnp.int32)
    dst3h = jnp.concatenate([dst_w, pad_d], axis=1).reshape(NW, CHP, KP)

    src_t = ei[0].reshape(NS, EPT)
    dst_t = ei[1].reshape(NS, EPT)
    pad_s2 = jnp.zeros((NS, EP2 - EPT), jnp.int32)
    pad_d2 = jnp.full((NS, EP2 - EPT), NP, jnp.int32)
    src2 = jnp.concatenate([src_t, pad_s2], axis=1).reshape(NS, CH2, KP)
    dst2 = jnp.concatenate([dst_t, pad_d2], axis=1)
    trash = QR + (jnp.arange(EP2, dtype=jnp.int32) % 8)[None, :]
    def _loc(p, c):
        loc = dst2 - (2 * p + c) * QR
        ok = (loc >= 0) & (loc < QR)
        return jnp.where(ok, loc, trash).reshape(NS, CH2, KP)
    dstq = jnp.stack(
        [jnp.stack([_loc(p, c) for c in range(NC)])
         for p in range(NQ // NC)])
    return dst3h, src2, dstq


def kernel(x, edge_index, W1, b1, W2, b2, W3, b3):
    ei = edge_index.astype(jnp.int32)
    dst3h, src2, dstq = _pad_worker_lists(ei)
    xp = jnp.pad(x, ((0, NP - N), (0, 0)))
    _sc_hist, _sc_agg = _build_sc_kernels()
    hist = _sc_hist(dst3h)
    hp1, dis = _pre_call(xp, W1, hist)
    p1 = _sc_agg(hp1, src2, dstq)
    hp2 = _mid_call(p1, hp1, dis, b1.reshape(1, D), W2)
    p2 = _sc_agg(hp2, src2, dstq)
    hp3 = _mid_call(p2, hp2, dis, b2.reshape(1, D), W3)
    p3 = _sc_agg(hp3, src2, dstq)
    return _post_call(p3, hp3, dis, b3.reshape(1, D))[:N]
